# baseline (speedup 1.0000x reference)
import functools

import jax
import jax.numpy as jnp
from jax import lax
from jax.experimental import pallas as pl
from jax.experimental.pallas import tpu as pltpu
from jax.experimental.pallas import tpu_sc as plsc

_VOCAB = 1000000
_EMBED = 64
_SEQ = 200
_BATCH = 4096

_NC = 2
_NS = 16
_NW = _NC * _NS
_BPW = _BATCH // _NW

_VT = 520000
_CHUNK = 400
_NCH_SC = (_VOCAB - _VT) // _CHUNK

_ROWS_PER_BLK = 8192


def _proj_body(emb_ref, w_ref, b_ref, out_ref):
    acc = lax.dot_general(
        w_ref[...], emb_ref[...],
        dimension_numbers=(((1,), (1,)), ((), ())),
        preferred_element_type=jnp.float32,
    )
    out_ref[...] = ((acc + b_ref[0]) * (1.0 / _SEQ)).reshape(_ROWS_PER_BLK)


def _project_table_tc(emb, W, b):
    grid = ((_VT + _ROWS_PER_BLK - 1) // _ROWS_PER_BLK,)
    return pl.pallas_call(
        _proj_body,
        grid=grid,
        in_specs=[
            pl.BlockSpec((_ROWS_PER_BLK, _EMBED), lambda i: (i, 0)),
            pl.BlockSpec((1, _EMBED), lambda i: (0, 0)),
            pl.BlockSpec(memory_space=pltpu.SMEM),
        ],
        out_specs=pl.BlockSpec((_ROWS_PER_BLK,), lambda i: (i,)),
        out_shape=jax.ShapeDtypeStruct((_VT,), jnp.float32),
    )(emb, W, b)


def _sc_proj_body(emb_hbm, wb_hbm, p_hbm, buf_v, wb_v, stage_v, out_v, sem_a, sem_b):
    wid = lax.axis_index("s") * _NC + lax.axis_index("c")
    pltpu.sync_copy(wb_hbm, wb_v)
    kpw = (_NCH_SC + _NW - 1) // _NW

    def fire(k, buf, sem):
        c = wid + k * _NW

        @pl.when(c < _NCH_SC)
        def _():
            pltpu.async_copy(
                emb_hbm.at[pl.ds(_VT + c * _CHUNK, _CHUNK), :],
                buf_v.at[buf], sem,
            )

    def wait(c, buf, sem):
        @pl.when(c < _NCH_SC)
        def _():
            pltpu.make_async_copy(
                emb_hbm.at[pl.ds(_VT, _CHUNK), :], buf_v.at[buf], sem
            ).wait()

    def compute_store(k, buf):
        c = wid + k * _NW

        @pl.when(c < _NCH_SC)
        def _():
            w_vecs = [wb_v[gg, pl.ds(0, 16)] for gg in range(_EMBED // 16)]
            b_hot = wb_v[_EMBED // 16, pl.ds(0, 16)]

            lane = lax.iota(jnp.int32, 16)
            last = jnp.full((16,), 15, jnp.int32)

            def group(g, carry):
                for i in range(16):
                    r = g * 16 + i
                    part = b_hot + buf_v[buf, r, pl.ds(0, 16)] * w_vecs[0]
                    for gg in range(1, _EMBED // 16):
                        part = part + (
                            buf_v[buf, r, pl.ds(gg * 16, 16)] * w_vecs[gg]
                        )
                    stage_v[i, pl.ds(0, 16)] = plsc.cumsum(part)
                tot = plsc.load_gather(stage_v, [lane, last])
                out_v[pl.ds(g * 16, 16)] = tot
                return carry

            lax.fori_loop(0, _CHUNK // 16, group, 0)
            pltpu.sync_copy(out_v, p_hbm.at[pl.ds(c * _CHUNK, _CHUNK)])

    fire(0, 0, sem_a)

    def pair(j, carry):
        k0 = 2 * j
        fire(k0 + 1, 1, sem_b)
        wait(wid + k0 * _NW, 0, sem_a)
        compute_store(k0, 0)
        fire(k0 + 2, 0, sem_a)
        wait(wid + (k0 + 1) * _NW, 1, sem_b)
        compute_store(k0 + 1, 1)
        return carry

    lax.fori_loop(0, (kpw + 1) // 2, pair, 0)


def _project_table_sc(emb, wb):
    mesh = plsc.VectorSubcoreMesh(core_axis_name="c", subcore_axis_name="s")
    fn = functools.partial(
        pl.kernel,
        mesh=mesh,
        out_type=jax.ShapeDtypeStruct((_VOCAB - _VT,), jnp.float32),
        scratch_types=[
            pltpu.VMEM((2, _CHUNK, _EMBED), jnp.float32),
            pltpu.VMEM((8, 16), jnp.float32),
            pltpu.VMEM((16, 16), jnp.float32),
            pltpu.VMEM((_CHUNK,), jnp.float32),
            pltpu.SemaphoreType.DMA,
            pltpu.SemaphoreType.DMA,
        ],
        compiler_params=pltpu.CompilerParams(needs_layout_passes=False),
    )(_sc_proj_body)
    return fn(emb, wb)


def _sc_body(x_hbm, p_hbm, out_hbm, idx_v, vals_v, out_v, sem):
    wid = lax.axis_index("s") * _NC + lax.axis_index("c")
    base = wid * _BPW
    pltpu.sync_copy(x_hbm.at[:, pl.ds(base, _BPW)], idx_v)
    chunk = 20

    def gather_chunk(c, carry):
        cps = [
            pltpu.async_copy(
                p_hbm.at[idx_v.at[c * chunk + j]], vals_v.at[c * chunk + j], sem
            )
            for j in range(chunk)
        ]
        for cp in cps:
            cp.wait()
        return carry

    lax.fori_loop(0, _SEQ // chunk, gather_chunk, 0)
    def step(s, accs):
        return tuple(
            accs[g] + vals_v[s, pl.ds(g * 16, 16)] for g in range(_BPW // 16)
        )
    accs = lax.fori_loop(
        0, _SEQ, step,
        tuple(jnp.zeros((16,), jnp.float32) for _ in range(_BPW // 16)),
    )
    for g in range(_BPW // 16):
        out_v[pl.ds(g * 16, 16)] = 1.0 / (1.0 + jnp.exp(-accs[g]))
    pltpu.sync_copy(out_v, out_hbm.at[pl.ds(base, _BPW)])


def _sc_pool(x, p_flat):
    mesh = plsc.VectorSubcoreMesh(core_axis_name="c", subcore_axis_name="s")
    fn = functools.partial(
        pl.kernel,
        mesh=mesh,
        out_type=jax.ShapeDtypeStruct((_BATCH,), jnp.float32),
        scratch_types=[
            pltpu.VMEM((_SEQ, _BPW), jnp.int32),
            pltpu.VMEM((_SEQ, _BPW), jnp.float32),
            pltpu.VMEM((_BPW,), jnp.float32),
            pltpu.SemaphoreType.DMA,
        ],
    )(_sc_body)
    return fn(x, p_flat)


def kernel(x, emb, W, b):
    wb = jnp.concatenate(
        [
            (W[0] * (1.0 / _SEQ)).reshape(_EMBED // 16, 16),
            jnp.zeros((1, 16), jnp.float32).at[0, 0].set(b[0] * (1.0 / _SEQ)),
            jnp.zeros((3, 16), jnp.float32),
        ],
        axis=0,
    ).astype(jnp.float32)
    p_tc = _project_table_tc(emb, W, b)
    p_sc = _project_table_sc(emb, wb)
    p = jnp.concatenate([p_tc, p_sc])
    out = _sc_pool(x, p)
    return out.reshape(_BATCH, 1)


# speedup vs baseline: 1.0000x; 1.0000x over previous
import functools

import jax
import jax.numpy as jnp
from jax import lax
from jax.experimental import pallas as pl
from jax.experimental.pallas import tpu as pltpu
from jax.experimental.pallas import tpu_sc as plsc

_VOCAB = 1000000
_EMBED = 64
_SEQ = 200
_BATCH = 4096

_NC = 2
_NS = 16
_NW = _NC * _NS
_BPW = _BATCH // _NW

_VT = 520000
_CHUNK = 400
_NCH_SC = (_VOCAB - _VT) // _CHUNK

_ROWS_PER_BLK = 8192


def _proj_body(emb_ref, w_ref, b_ref, out_ref):
    acc = lax.dot_general(
        w_ref[...], emb_ref[...],
        dimension_numbers=(((1,), (1,)), ((), ())),
        preferred_element_type=jnp.float32,
    )
    out_ref[...] = ((acc + b_ref[0]) * (1.0 / _SEQ)).reshape(_ROWS_PER_BLK)


def _project_table_tc(emb, W, b):
    grid = ((_VT + _ROWS_PER_BLK - 1) // _ROWS_PER_BLK,)
    return pl.pallas_call(
        _proj_body,
        grid=grid,
        in_specs=[
            pl.BlockSpec((_ROWS_PER_BLK, _EMBED), lambda i: (i, 0)),
            pl.BlockSpec((1, _EMBED), lambda i: (0, 0)),
            pl.BlockSpec(memory_space=pltpu.SMEM),
        ],
        out_specs=pl.BlockSpec((_ROWS_PER_BLK,), lambda i: (i,)),
        out_shape=jax.ShapeDtypeStruct((_VT,), jnp.float32),
    )(emb, W, b)


def _sc_proj_body(emb_hbm, wb_hbm, p_hbm, buf_v, wb_v, stage_v, out_v, sem_a, sem_b):
    wid = lax.axis_index("s") * _NC + lax.axis_index("c")
    pltpu.sync_copy(wb_hbm, wb_v)
    kpw = (_NCH_SC + _NW - 1) // _NW

    def fire(k, buf, sem):
        c = wid + k * _NW

        @pl.when(c < _NCH_SC)
        def _():
            pltpu.async_copy(
                emb_hbm.at[pl.ds(_VT + c * _CHUNK, _CHUNK), :],
                buf_v.at[buf], sem,
            )

    def wait(c, buf, sem):
        @pl.when(c < _NCH_SC)
        def _():
            pltpu.make_async_copy(
                emb_hbm.at[pl.ds(_VT, _CHUNK), :], buf_v.at[buf], sem
            ).wait()

    def compute_store(k, buf):
        c = wid + k * _NW

        @pl.when(c < _NCH_SC)
        def _():
            w_vecs = [wb_v[gg, pl.ds(0, 16)] for gg in range(_EMBED // 16)]
            b_hot = wb_v[_EMBED // 16, pl.ds(0, 16)]

            lane = lax.iota(jnp.int32, 16)
            last = jnp.full((16,), 15, jnp.int32)

            def group(g, carry):
                for i in range(16):
                    r = g * 16 + i
                    part = b_hot + buf_v[buf, r, pl.ds(0, 16)] * w_vecs[0]
                    for gg in range(1, _EMBED // 16):
                        part = part + (
                            buf_v[buf, r, pl.ds(gg * 16, 16)] * w_vecs[gg]
                        )
                    stage_v[i, pl.ds(0, 16)] = plsc.cumsum(part)
                tot = plsc.load_gather(stage_v, [lane, last])
                out_v[pl.ds(g * 16, 16)] = tot
                return carry

            lax.fori_loop(0, _CHUNK // 16, group, 0)
            pltpu.sync_copy(out_v, p_hbm.at[pl.ds(c * _CHUNK, _CHUNK)])

    fire(0, 0, sem_a)

    def pair(j, carry):
        k0 = 2 * j
        fire(k0 + 1, 1, sem_b)
        wait(wid + k0 * _NW, 0, sem_a)
        compute_store(k0, 0)
        fire(k0 + 2, 0, sem_a)
        wait(wid + (k0 + 1) * _NW, 1, sem_b)
        compute_store(k0 + 1, 1)
        return carry

    lax.fori_loop(0, (kpw + 1) // 2, pair, 0)


def _project_table_sc(emb, wb):
    mesh = plsc.VectorSubcoreMesh(core_axis_name="c", subcore_axis_name="s")
    fn = functools.partial(
        pl.kernel,
        mesh=mesh,
        out_type=jax.ShapeDtypeStruct((_VOCAB - _VT,), jnp.float32),
        scratch_types=[
            pltpu.VMEM((2, _CHUNK, _EMBED), jnp.float32),
            pltpu.VMEM((8, 16), jnp.float32),
            pltpu.VMEM((16, 16), jnp.float32),
            pltpu.VMEM((_CHUNK,), jnp.float32),
            pltpu.SemaphoreType.DMA,
            pltpu.SemaphoreType.DMA,
        ],
        compiler_params=pltpu.CompilerParams(needs_layout_passes=False),
    )(_sc_proj_body)
    return fn(emb, wb)


def _sc_body(x_hbm, p_hbm, out_hbm, idx_v, vals_v, out_v, sem):
    wid = lax.axis_index("s") * _NC + lax.axis_index("c")
    base = wid * _BPW
    pltpu.sync_copy(x_hbm.at[:, pl.ds(base, _BPW)], idx_v)
    chunk = 20

    def gather_chunk(c, carry):
        cps = [
            pltpu.async_copy(
                p_hbm.at[idx_v.at[c * chunk + j]], vals_v.at[c * chunk + j], sem
            )
            for j in range(chunk)
        ]
        for cp in cps:
            cp.wait()
        return carry

    lax.fori_loop(0, _SEQ // chunk, gather_chunk, 0)
    def step(s, accs):
        return tuple(
            accs[g] + vals_v[s, pl.ds(g * 16, 16)] for g in range(_BPW // 16)
        )
    accs = lax.fori_loop(
        0, _SEQ, step,
        tuple(jnp.zeros((16,), jnp.float32) for _ in range(_BPW // 16)),
    )
    for g in range(_BPW // 16):
        out_v[pl.ds(g * 16, 16)] = 1.0 / (1.0 + jnp.exp(-accs[g]))
    pltpu.sync_copy(out_v, out_hbm.at[pl.ds(base, _BPW)])


def _sc_pool(x, p_flat):
    mesh = plsc.VectorSubcoreMesh(core_axis_name="c", subcore_axis_name="s")
    fn = functools.partial(
        pl.kernel,
        mesh=mesh,
        out_type=jax.ShapeDtypeStruct((_BATCH,), jnp.float32),
        scratch_types=[
            pltpu.VMEM((_SEQ, _BPW), jnp.int32),
            pltpu.VMEM((_SEQ, _BPW), jnp.float32),
            pltpu.VMEM((_BPW,), jnp.float32),
            pltpu.SemaphoreType.DMA,
        ],
    )(_sc_body)
    return fn(x, p_flat)


def kernel(x, emb, W, b):
    wb = jnp.concatenate(
        [
            (W[0] * (1.0 / _SEQ)).reshape(_EMBED // 16, 16),
            jnp.zeros((1, 16), jnp.float32).at[0, 0].set(b[0] * (1.0 / _SEQ)),
            jnp.zeros((3, 16), jnp.float32),
        ],
        axis=0,
    ).astype(jnp.float32)
    p_sc = _project_table_sc(emb, wb)
    p_tc = _project_table_tc(emb, W, b)
    p = jnp.concatenate([p_tc, p_sc])
    out = _sc_pool(x, p)
    return out.reshape(_BATCH, 1)


# speedup vs baseline: 1.2212x; 1.2212x over previous
import functools

import jax
import jax.numpy as jnp
from jax import lax
from jax.experimental import pallas as pl
from jax.experimental.pallas import tpu as pltpu
from jax.experimental.pallas import tpu_sc as plsc

_VOCAB = 1000000
_EMBED = 64
_SEQ = 200
_BATCH = 4096

_ROWS_PER_BLK = 32768


def _proj_body(emb_ref, w_ref, b_ref, out_ref):
    acc = lax.dot_general(
        w_ref[...], emb_ref[...],
        dimension_numbers=(((1,), (1,)), ((), ())),
        preferred_element_type=jnp.float32,
    )
    out_ref[...] = ((acc + b_ref[0]) * (1.0 / _SEQ)).reshape(_ROWS_PER_BLK)


def _project_table(emb, W, b):
    grid = ((_VOCAB + _ROWS_PER_BLK - 1) // _ROWS_PER_BLK,)
    return pl.pallas_call(
        _proj_body,
        grid=grid,
        in_specs=[
            pl.BlockSpec((_ROWS_PER_BLK, _EMBED), lambda i: (i, 0)),
            pl.BlockSpec((1, _EMBED), lambda i: (0, 0)),
            pl.BlockSpec(memory_space=pltpu.SMEM),
        ],
        out_specs=pl.BlockSpec((_ROWS_PER_BLK,), lambda i: (i,)),
        out_shape=jax.ShapeDtypeStruct((_VOCAB,), jnp.float32),
    )(emb, W, b)


_NC = 2
_NS = 16
_NW = _NC * _NS
_BPW = _BATCH // _NW


def _sc_body(x_hbm, p_hbm, out_hbm, idx_v, vals_v, out_v, sem_a, sem_b):
    wid = lax.axis_index("s") * _NC + lax.axis_index("c")
    base = wid * _BPW
    pltpu.sync_copy(x_hbm.at[:, pl.ds(base, _BPW)], idx_v)
    chunk = 10

    def fire(c, sem):
        return [
            pltpu.async_copy(
                p_hbm.at[idx_v.at[c * chunk + j]], vals_v.at[c * chunk + j], sem
            )
            for j in range(chunk)
        ]

    def drain(c, sem):
        for j in range(chunk):
            pltpu.make_async_copy(
                p_hbm.at[idx_v.at[j]], vals_v.at[c * chunk + j], sem
            ).wait()

    fire(0, sem_a)

    def gather_pair(h, carry):
        c0 = 2 * h
        fire(c0 + 1, sem_b)
        drain(c0, sem_a)

        @pl.when(c0 + 2 < _SEQ // chunk)
        def _():
            fire(c0 + 2, sem_a)

        drain(c0 + 1, sem_b)
        return carry

    lax.fori_loop(0, _SEQ // chunk // 2, gather_pair, 0)
    def step(s, accs):
        return tuple(
            accs[g] + vals_v[s, pl.ds(g * 16, 16)] for g in range(_BPW // 16)
        )
    accs = lax.fori_loop(
        0, _SEQ, step,
        tuple(jnp.zeros((16,), jnp.float32) for _ in range(_BPW // 16)),
    )
    for g in range(_BPW // 16):
        out_v[pl.ds(g * 16, 16)] = 1.0 / (1.0 + jnp.exp(-accs[g]))
    pltpu.sync_copy(out_v, out_hbm.at[pl.ds(base, _BPW)])


def _sc_pool(x, p_flat):
    mesh = plsc.VectorSubcoreMesh(core_axis_name="c", subcore_axis_name="s")
    fn = functools.partial(
        pl.kernel,
        mesh=mesh,
        out_type=jax.ShapeDtypeStruct((_BATCH,), jnp.float32),
        scratch_types=[
            pltpu.VMEM((_SEQ, _BPW), jnp.int32),
            pltpu.VMEM((_SEQ, _BPW), jnp.float32),
            pltpu.VMEM((_BPW,), jnp.float32),
            pltpu.SemaphoreType.DMA,
            pltpu.SemaphoreType.DMA,
        ],
    )(_sc_body)
    return fn(x, p_flat)


def kernel(x, emb, W, b):
    p = _project_table(emb, W, b)
    out = _sc_pool(x, p)
    return out.reshape(_BATCH, 1)
